# Initial kernel scaffold; baseline (speedup 1.0000x reference)
import jax
import jax.numpy as jnp
from jax.experimental import pallas as pl


def kernel(x, edge_index, W1, b1, W2, b2):
    raise NotImplementedError("write your pallas kernel here")


# speedup vs baseline: 25.9026x; 25.9026x over previous
import functools

import jax
import jax.numpy as jnp
from jax import lax
from jax.experimental import pallas as pl
from jax.experimental.pallas import tpu as pltpu
from jax.experimental.pallas import tpu_sc as plsc

N = 10000
NP = 10240
D = 128
H = 16
CH = 128
NW = 32
STRIPE = NP // 16

_mesh = plsc.VectorSubcoreMesh(core_axis_name="c", subcore_axis_name="s")


def _worker(c, s):
    return s * 2 + c


def _sc_deg_body(n_per_w, edges, zeros, out, idx_v, ones_v, acc_sh, sem):
    c = lax.axis_index("c")
    s = lax.axis_index("s")
    w = _worker(c, s)

    @pl.when(s == 0)
    def _():
        pltpu.sync_copy(zeros, acc_sh)

    for j in range(CH // 16):
        ones_v[pl.ds(j * 16, 16)] = jnp.ones((16,), jnp.float32)
    plsc.subcore_barrier()

    def body(i, carry):
        chunk = w * n_per_w + i
        pltpu.sync_copy(edges.at[chunk], idx_v)
        pltpu.sync_copy(ones_v, acc_sh.at[idx_v.at[1]], add=True)
        return carry

    lax.fori_loop(0, n_per_w, body, 0)
    plsc.subcore_barrier()
    off = pl.multiple_of(c * NP + s * STRIPE, STRIPE)
    pltpu.sync_copy(acc_sh.at[pl.ds(s * STRIPE, STRIPE)],
                    out.at[pl.ds(off, STRIPE)])


def _sc_deg(edges, zeros):
    n_per_w = edges.shape[0] // NW
    return pl.kernel(
        functools.partial(_sc_deg_body, n_per_w),
        mesh=_mesh,
        compiler_params=pltpu.CompilerParams(use_tc_tiling_on_sc=False),
        out_type=jax.ShapeDtypeStruct((2 * NP,), jnp.float32),
        scratch_types=[
            pltpu.VMEM((2, CH), jnp.int32),
            pltpu.VMEM((CH,), jnp.float32),
            pltpu.VMEM_SHARED((NP,), jnp.float32),
            pltpu.SemaphoreType.DMA,
        ],
    )(edges, zeros)


def _sc_agg16_body(n_per_w, edges, table, zeros, out, idx_v, rows_v, acc_sh, sem):
    c = lax.axis_index("c")
    s = lax.axis_index("s")
    w = _worker(c, s)

    @pl.when(s == 0)
    def _():
        pltpu.sync_copy(zeros, acc_sh)

    plsc.subcore_barrier()

    def body(i, carry):
        chunk = w * n_per_w + i
        pltpu.sync_copy(edges.at[chunk], idx_v)
        pltpu.async_copy(table.at[idx_v.at[0]], rows_v, sem).wait()
        pltpu.sync_copy(rows_v, acc_sh.at[idx_v.at[1]], add=True)
        return carry

    lax.fori_loop(0, n_per_w, body, 0)
    plsc.subcore_barrier()
    off = pl.multiple_of(c * NP + s * STRIPE, STRIPE)
    pltpu.sync_copy(acc_sh.at[pl.ds(s * STRIPE, STRIPE)],
                    out.at[pl.ds(off, STRIPE)])


def _sc_agg16(edges, table, zeros):
    n_per_w = edges.shape[0] // NW
    return pl.kernel(
        functools.partial(_sc_agg16_body, n_per_w),
        mesh=_mesh,
        compiler_params=pltpu.CompilerParams(use_tc_tiling_on_sc=False),
        out_type=jax.ShapeDtypeStruct((2 * NP, H), jnp.float32),
        scratch_types=[
            pltpu.VMEM((2, CH), jnp.int32),
            pltpu.VMEM((CH, H), jnp.float32),
            pltpu.VMEM_SHARED((NP, H), jnp.float32),
            pltpu.SemaphoreType.DMA,
        ],
    )(edges, table, zeros)


def _sc_agg1_body(n_per_w, edges, table, zeros, out, idx_v, vals_v, acc_sh, sem):
    c = lax.axis_index("c")
    s = lax.axis_index("s")
    w = _worker(c, s)

    @pl.when(s == 0)
    def _():
        pltpu.sync_copy(zeros, acc_sh)

    plsc.subcore_barrier()

    def body(i, carry):
        chunk = w * n_per_w + i
        pltpu.sync_copy(edges.at[chunk], idx_v)
        pltpu.async_copy(table.at[idx_v.at[0]], vals_v, sem).wait()
        pltpu.sync_copy(vals_v, acc_sh.at[idx_v.at[1]], add=True)
        return carry

    lax.fori_loop(0, n_per_w, body, 0)
    plsc.subcore_barrier()
    off = pl.multiple_of(c * NP + s * STRIPE, STRIPE)
    pltpu.sync_copy(acc_sh.at[pl.ds(s * STRIPE, STRIPE)],
                    out.at[pl.ds(off, STRIPE)])


def _sc_agg1(edges, table, zeros):
    n_per_w = edges.shape[0] // NW
    return pl.kernel(
        functools.partial(_sc_agg1_body, n_per_w),
        mesh=_mesh,
        compiler_params=pltpu.CompilerParams(use_tc_tiling_on_sc=False),
        out_type=jax.ShapeDtypeStruct((2 * NP,), jnp.float32),
        scratch_types=[
            pltpu.VMEM((2, CH), jnp.int32),
            pltpu.VMEM((CH,), jnp.float32),
            pltpu.VMEM_SHARED((NP,), jnp.float32),
            pltpu.SemaphoreType.DMA,
        ],
    )(edges, table, zeros)


def _tc_stage1_body(x_ref, w1_ref, dA_ref, dB_ref, dinv_ref, hs1_ref):
    deg = dA_ref[...] + dB_ref[...] + 1.0
    dinv = lax.rsqrt(deg)
    h = jnp.dot(x_ref[...], w1_ref[...], preferred_element_type=jnp.float32)
    dinv_ref[...] = dinv
    hs1_ref[...] = h * dinv


def _tc_stage1(x, w1, degA, degB):
    return pl.pallas_call(
        _tc_stage1_body,
        out_shape=(
            jax.ShapeDtypeStruct((N, 1), jnp.float32),
            jax.ShapeDtypeStruct((N, H), jnp.float32),
        ),
    )(x, w1, degA, degB)


def _tc_stage2_body(a1A_ref, a1B_ref, hs1_ref, dinv_ref, b1_ref, w2_ref, hs2_ref):
    dinv = dinv_ref[...]
    pre = dinv * (a1A_ref[...] + a1B_ref[...] + hs1_ref[...]) + b1_ref[...]
    y1 = jnp.maximum(pre, 0.0)
    h2 = jnp.dot(y1, w2_ref[...], preferred_element_type=jnp.float32)
    hs2_ref[...] = h2 * dinv


def _tc_stage2(acc1A, acc1B, hs1, dinv, b1, w2):
    return pl.pallas_call(
        _tc_stage2_body,
        out_shape=jax.ShapeDtypeStruct((N, 1), jnp.float32),
    )(acc1A, acc1B, hs1, dinv, b1, w2)


def _tc_stage3_body(a2A_ref, a2B_ref, hs2_ref, dinv_ref, b2_ref, out_ref):
    out_ref[...] = (dinv_ref[...] * (a2A_ref[...] + a2B_ref[...] + hs2_ref[...])
                    + b2_ref[...])


def _tc_stage3(acc2A, acc2B, hs2, dinv, b2):
    return pl.pallas_call(
        _tc_stage3_body,
        out_shape=jax.ShapeDtypeStruct((N, 1), jnp.float32),
    )(acc2A, acc2B, hs2, dinv, b2)


@jax.jit
def kernel(x, edge_index, W1, b1, W2, b2):
    E = edge_index.shape[1]
    n_per_w = pl.cdiv(pl.cdiv(E, CH), NW)
    E_pad = NW * n_per_w * CH
    src = jnp.concatenate(
        [edge_index[0], jnp.zeros((E_pad - E,), jnp.int32)])
    dst = jnp.concatenate(
        [edge_index[1], jnp.full((E_pad - E,), N, jnp.int32)])
    edges = jnp.stack([src, dst]).reshape(2, -1, CH).transpose(1, 0, 2)

    zeros16 = jnp.zeros((NP, H), jnp.float32)
    zeros1 = jnp.zeros((NP,), jnp.float32)

    deg = _sc_deg(edges, zeros1)
    degA = deg[:N, None]
    degB = deg[NP:NP + N, None]

    dinv, hs1 = _tc_stage1(x, W1, degA, degB)

    acc1 = _sc_agg16(edges, hs1, zeros16)

    hs2 = _tc_stage2(acc1[:N], acc1[NP:NP + N], hs1, dinv,
                     b1.reshape(1, H), W2)

    acc2 = _sc_agg1(edges, hs2.reshape(N), zeros1)

    out = _tc_stage3(acc2[:N, None], acc2[NP:NP + N, None], hs2, dinv,
                     b2.reshape(1, 1))
    return out
